# Initial kernel scaffold; baseline (speedup 1.0000x reference)
import jax
import jax.numpy as jnp
from jax.experimental import pallas as pl


def kernel(prediction):
    raise NotImplementedError("write your pallas kernel here")


# speedup vs baseline: 17.5793x; 17.5793x over previous
import jax
import jax.numpy as jnp
from jax.experimental import pallas as pl

_CONF = 0.25
_IOU = 0.45
_MAXDET = 300
_MAXWH = 4096.0
_K = 2048
_NEG = -1e9
_NPAD = 20480
_ACHUNK = 10240
_BS = 512
_NBLK = _K // _BS


def _score_body(p_ref, o_ref):
    xc = p_ref[0, :, 0:1]
    yc = p_ref[0, :, 1:2]
    w = p_ref[0, :, 2:3]
    h = p_ref[0, :, 3:4]
    obj = p_ref[0, :, 4:5]
    cls = p_ref[0, :, 5:85]
    scaled = cls * obj
    conf = jnp.max(scaled, axis=1, keepdims=True)
    iota = jax.lax.broadcasted_iota(jnp.int32, scaled.shape, 1)
    j = jnp.min(jnp.where(scaled == conf, iota, 1000), axis=1, keepdims=True)
    b0 = xc - w / 2
    b1 = yc - h / 2
    b2 = xc + w / 2
    b3 = yc + h / 2
    cand = (obj > _CONF) & (conf > _CONF)
    score = jnp.where(cand, conf, _NEG)
    zero = jnp.zeros_like(score)
    out = jnp.concatenate(
        [b0, b1, b2, b3, score, j.astype(jnp.float32), zero, zero], axis=1)
    o_ref[...] = out[None]


def _nms_body(sel_ref, selT_ref, ks_ref):
    s = sel_ref[0]
    sT = selT_ref[0]
    cls_r = s[:, 5:6]
    br = s[:, 0:4] + cls_r * _MAXWH
    cls_c = sT[5:6, :] * _MAXWH
    x0c = sT[0:1, :] + cls_c
    y0c = sT[1:2, :] + cls_c
    x1c = sT[2:3, :] + cls_c
    y1c = sT[3:4, :] + cls_c
    score_c = sT[4:5, :]
    area_c = (x1c - x0c) * (y1c - y0c)
    valid_c = score_c > (_NEG / 2)
    col = jax.lax.broadcasted_iota(jnp.int32, (1, _K), 1)

    sup = jnp.zeros((1, _K), dtype=jnp.bool_)
    keep_parts = []
    for b in range(_NBLK):
        r0 = b * _BS
        x0r = br[r0:r0 + _BS, 0:1]
        y0r = br[r0:r0 + _BS, 1:2]
        x1r = br[r0:r0 + _BS, 2:3]
        y1r = br[r0:r0 + _BS, 3:4]
        area_r = (x1r - x0r) * (y1r - y0r)
        ltx = jnp.maximum(x0r, x0c)
        lty = jnp.maximum(y0r, y0c)
        rbx = jnp.minimum(x1r, x1c)
        rby = jnp.minimum(y1r, y1c)
        iw = jnp.clip(rbx - ltx, 0.0, None)
        ih = jnp.clip(rby - lty, 0.0, None)
        inter = iw * ih
        iou = inter / (area_r + area_c - inter + 1e-9)
        row = jax.lax.broadcasted_iota(jnp.int32, (_BS, 1), 0) + r0
        m = jnp.where((iou > _IOU) & (col > row), 1.0, 0.0)
        mbb = m[:, r0:r0 + _BS]
        validb = valid_c[:, r0:r0 + _BS] & jnp.logical_not(sup[:, r0:r0 + _BS])
        vbf = jnp.where(validb, 1.0, 0.0)

        def _cond(c):
            return c[1]

        def _body(c):
            k, _ = c
            prod = jax.lax.dot_general(
                k, mbb, (((1,), (0,)), ((), ())),
                preferred_element_type=jnp.float32)
            knew = jnp.where(prod == 0.0, vbf, 0.0)
            return (knew, jnp.any(knew != k))

        kb, _ = jax.lax.while_loop(_cond, _body, (vbf, jnp.bool_(True)))
        prod_all = jax.lax.dot_general(
            kb, m, (((1,), (0,)), ((), ())),
            preferred_element_type=jnp.float32)
        sup = sup | (prod_all > 0.0)
        keep_parts.append(kb > 0.0)

    keep = jnp.concatenate(keep_parts, axis=1)
    ks = jnp.where(keep, score_c, _NEG)
    ks_ref[...] = ks[None]


def kernel(prediction):
    B, N, C = prediction.shape
    pred_p = jnp.pad(prediction, ((0, 0), (0, _NPAD - N), (0, 0)))
    feat = pl.pallas_call(
        _score_body,
        grid=(B, _NPAD // _ACHUNK),
        in_specs=[pl.BlockSpec((1, _ACHUNK, C), lambda i, a: (i, a, 0))],
        out_specs=pl.BlockSpec((1, _ACHUNK, 8), lambda i, a: (i, a, 0)),
        out_shape=jax.ShapeDtypeStruct((B, _NPAD, 8), jnp.float32),
    )(pred_p)

    scores = feat[:, :, 4]
    _, idx = jax.lax.top_k(scores, _K)
    sel = jnp.take_along_axis(feat, idx[:, :, None], axis=1)
    selT = jnp.transpose(sel, (0, 2, 1))

    ks = pl.pallas_call(
        _nms_body,
        grid=(B,),
        in_specs=[
            pl.BlockSpec((1, _K, 8), lambda i: (i, 0, 0)),
            pl.BlockSpec((1, 8, _K), lambda i: (i, 0, 0)),
        ],
        out_specs=pl.BlockSpec((1, 1, _K), lambda i: (i, 0, 0)),
        out_shape=jax.ShapeDtypeStruct((B, 1, _K), jnp.float32),
    )(sel, selT)[:, 0, :]

    det_scores, det_idx = jax.lax.top_k(ks, _MAXDET)
    det = jnp.take_along_axis(sel, det_idx[:, :, None], axis=1)
    mask = det_scores > (_NEG / 2)
    out = jnp.concatenate(
        [det[..., 0:4], det_scores[..., None], det[..., 5:6]], axis=2)
    out = jnp.where(mask[..., None], out, 0.0)
    counts = mask.sum(axis=1)
    return out, counts


# speedup vs baseline: 21.7465x; 1.2371x over previous
import jax
import jax.numpy as jnp
from jax.experimental import pallas as pl

_CONF = 0.25
_IOU = 0.45
_MAXDET = 300
_MAXWH = 4096.0
_K = 2048
_NEG = -1e9
_ACHUNK = 2000
_BS = 512
_NBLK = _K // _BS


def _score_body(p_ref, o_ref):
    xc = p_ref[0, :, 0:1]
    yc = p_ref[0, :, 1:2]
    w = p_ref[0, :, 2:3]
    h = p_ref[0, :, 3:4]
    obj = p_ref[0, :, 4:5]
    cls = p_ref[0, :, 5:85]
    scaled = cls * obj
    conf = jnp.max(scaled, axis=1, keepdims=True)
    iota = jax.lax.broadcasted_iota(jnp.int32, scaled.shape, 1)
    j = jnp.min(jnp.where(scaled == conf, iota, 1000), axis=1, keepdims=True)
    b0 = xc - w / 2
    b1 = yc - h / 2
    b2 = xc + w / 2
    b3 = yc + h / 2
    cand = (obj > _CONF) & (conf > _CONF)
    score = jnp.where(cand, conf, _NEG)
    zero = jnp.zeros_like(score)
    out = jnp.concatenate(
        [b0, b1, b2, b3, score, j.astype(jnp.float32), zero, zero], axis=1)
    o_ref[...] = out[None]


def _nms_body(sel_ref, selT_ref, ks_ref):
    s = sel_ref[0]
    sT = selT_ref[0]
    cls_r = s[:, 5:6]
    br = s[:, 0:4] + cls_r * _MAXWH
    cls_c = sT[5:6, :] * _MAXWH
    x0c = sT[0:1, :] + cls_c
    y0c = sT[1:2, :] + cls_c
    x1c = sT[2:3, :] + cls_c
    y1c = sT[3:4, :] + cls_c
    score_c = sT[4:5, :]
    area_c = (x1c - x0c) * (y1c - y0c)
    valid_c = score_c > (_NEG / 2)
    col = jax.lax.broadcasted_iota(jnp.int32, (1, _K), 1)

    sup = jnp.zeros((1, _K), dtype=jnp.bool_)
    keep_parts = []
    for b in range(_NBLK):
        r0 = b * _BS
        x0r = br[r0:r0 + _BS, 0:1]
        y0r = br[r0:r0 + _BS, 1:2]
        x1r = br[r0:r0 + _BS, 2:3]
        y1r = br[r0:r0 + _BS, 3:4]
        area_r = (x1r - x0r) * (y1r - y0r)
        ltx = jnp.maximum(x0r, x0c)
        lty = jnp.maximum(y0r, y0c)
        rbx = jnp.minimum(x1r, x1c)
        rby = jnp.minimum(y1r, y1c)
        iw = jnp.clip(rbx - ltx, 0.0, None)
        ih = jnp.clip(rby - lty, 0.0, None)
        inter = iw * ih
        iou = inter / (area_r + area_c - inter + 1e-9)
        row = jax.lax.broadcasted_iota(jnp.int32, (_BS, 1), 0) + r0
        m = jnp.where((iou > _IOU) & (col > row), 1.0, 0.0)
        mbb = m[:, r0:r0 + _BS]
        validb = valid_c[:, r0:r0 + _BS] & jnp.logical_not(sup[:, r0:r0 + _BS])
        vbf = jnp.where(validb, 1.0, 0.0)

        def _cond(c):
            return c[1]

        def _body(c):
            k, _ = c
            prod = jax.lax.dot_general(
                k, mbb, (((1,), (0,)), ((), ())),
                preferred_element_type=jnp.float32)
            knew = jnp.where(prod == 0.0, vbf, 0.0)
            return (knew, jnp.any(knew != k))

        kb, _ = jax.lax.while_loop(_cond, _body, (vbf, jnp.bool_(True)))
        prod_all = jax.lax.dot_general(
            kb, m, (((1,), (0,)), ((), ())),
            preferred_element_type=jnp.float32)
        sup = sup | (prod_all > 0.0)
        keep_parts.append(kb > 0.0)

    keep = jnp.concatenate(keep_parts, axis=1)
    ks = jnp.where(keep, score_c, _NEG)
    ks_ref[...] = ks[None]


def kernel(prediction):
    B, N, C = prediction.shape
    feat = pl.pallas_call(
        _score_body,
        grid=(B, N // _ACHUNK),
        in_specs=[pl.BlockSpec((1, _ACHUNK, C), lambda i, a: (i, a, 0))],
        out_specs=pl.BlockSpec((1, _ACHUNK, 8), lambda i, a: (i, a, 0)),
        out_shape=jax.ShapeDtypeStruct((B, N, 8), jnp.float32),
    )(prediction)

    scores = feat[:, :, 4]
    _, idx = jax.lax.top_k(scores, _K)
    sel = jnp.take_along_axis(feat, idx[:, :, None], axis=1)
    selT = jnp.transpose(sel, (0, 2, 1))

    ks = pl.pallas_call(
        _nms_body,
        grid=(B,),
        in_specs=[
            pl.BlockSpec((1, _K, 8), lambda i: (i, 0, 0)),
            pl.BlockSpec((1, 8, _K), lambda i: (i, 0, 0)),
        ],
        out_specs=pl.BlockSpec((1, 1, _K), lambda i: (i, 0, 0)),
        out_shape=jax.ShapeDtypeStruct((B, 1, _K), jnp.float32),
    )(sel, selT)[:, 0, :]

    det_scores, det_idx = jax.lax.top_k(ks, _MAXDET)
    det = jnp.take_along_axis(sel, det_idx[:, :, None], axis=1)
    mask = det_scores > (_NEG / 2)
    out = jnp.concatenate(
        [det[..., 0:4], det_scores[..., None], det[..., 5:6]], axis=2)
    out = jnp.where(mask[..., None], out, 0.0)
    counts = mask.sum(axis=1)
    return out, counts


# speedup vs baseline: 32.6660x; 1.5021x over previous
import jax
import jax.numpy as jnp
from jax.experimental import pallas as pl

_CONF = 0.25
_IOU = 0.45
_MAXDET = 300
_MAXWH = 4096.0
_K = 2048
_NEG = -1e9
_BS = 512
_NBLK = _K // _BS


def _score_body(p_ref, o_ref):
    xc = p_ref[0, 0:1, :]
    yc = p_ref[0, 1:2, :]
    w = p_ref[0, 2:3, :]
    h = p_ref[0, 3:4, :]
    obj = p_ref[0, 4:5, :]
    cls = p_ref[0, 5:85, :]
    scaled = cls * obj
    conf = jnp.max(scaled, axis=0, keepdims=True)
    iota = jax.lax.broadcasted_iota(jnp.int32, scaled.shape, 0)
    j = jnp.min(jnp.where(scaled == conf, iota, 1000), axis=0, keepdims=True)
    b0 = xc - w / 2
    b1 = yc - h / 2
    b2 = xc + w / 2
    b3 = yc + h / 2
    cand = (obj > _CONF) & (conf > _CONF)
    score = jnp.where(cand, conf, _NEG)
    zero = jnp.zeros_like(score)
    out = jnp.concatenate(
        [b0, b1, b2, b3, score, j.astype(jnp.float32), zero, zero], axis=0)
    o_ref[...] = out[None]


def _nms_body(sel_ref, selT_ref, ks_ref):
    s = sel_ref[0]
    sT = selT_ref[0]
    cls_r = s[:, 5:6]
    br = s[:, 0:4] + cls_r * _MAXWH
    cls_c = sT[5:6, :] * _MAXWH
    x0c = sT[0:1, :] + cls_c
    y0c = sT[1:2, :] + cls_c
    x1c = sT[2:3, :] + cls_c
    y1c = sT[3:4, :] + cls_c
    score_c = sT[4:5, :]
    area_c = (x1c - x0c) * (y1c - y0c)
    valid_c = score_c > (_NEG / 2)
    col = jax.lax.broadcasted_iota(jnp.int32, (1, _K), 1)

    sup = jnp.zeros((1, _K), dtype=jnp.bool_)
    keep_parts = []
    for b in range(_NBLK):
        r0 = b * _BS
        x0r = br[r0:r0 + _BS, 0:1]
        y0r = br[r0:r0 + _BS, 1:2]
        x1r = br[r0:r0 + _BS, 2:3]
        y1r = br[r0:r0 + _BS, 3:4]
        area_r = (x1r - x0r) * (y1r - y0r)
        ltx = jnp.maximum(x0r, x0c)
        lty = jnp.maximum(y0r, y0c)
        rbx = jnp.minimum(x1r, x1c)
        rby = jnp.minimum(y1r, y1c)
        iw = jnp.clip(rbx - ltx, 0.0, None)
        ih = jnp.clip(rby - lty, 0.0, None)
        inter = iw * ih
        iou = inter / (area_r + area_c - inter + 1e-9)
        row = jax.lax.broadcasted_iota(jnp.int32, (_BS, 1), 0) + r0
        m = jnp.where((iou > _IOU) & (col > row), 1.0, 0.0)
        mbb = m[:, r0:r0 + _BS]
        validb = valid_c[:, r0:r0 + _BS] & jnp.logical_not(sup[:, r0:r0 + _BS])
        vbf = jnp.where(validb, 1.0, 0.0)

        def _cond(c):
            return c[1]

        def _body(c):
            k, _ = c
            prod = jax.lax.dot_general(
                k, mbb, (((1,), (0,)), ((), ())),
                preferred_element_type=jnp.float32)
            knew = jnp.where(prod == 0.0, vbf, 0.0)
            return (knew, jnp.any(knew != k))

        kb, _ = jax.lax.while_loop(_cond, _body, (vbf, jnp.bool_(True)))
        prod_all = jax.lax.dot_general(
            kb, m, (((1,), (0,)), ((), ())),
            preferred_element_type=jnp.float32)
        sup = sup | (prod_all > 0.0)
        keep_parts.append(kb > 0.0)

    keep = jnp.concatenate(keep_parts, axis=1)
    ks = jnp.where(keep, score_c, _NEG)
    ks_ref[...] = ks[None]


def kernel(prediction):
    B, N, C = prediction.shape
    pt = jnp.transpose(prediction, (0, 2, 1))
    featT = pl.pallas_call(
        _score_body,
        grid=(B,),
        in_specs=[pl.BlockSpec((1, C, N), lambda i: (i, 0, 0))],
        out_specs=pl.BlockSpec((1, 8, N), lambda i: (i, 0, 0)),
        out_shape=jax.ShapeDtypeStruct((B, 8, N), jnp.float32),
    )(pt)

    scores = featT[:, 4, :]
    _, idx = jax.lax.top_k(scores, _K)
    selT = jnp.take_along_axis(featT, idx[:, None, :], axis=2)
    sel = jnp.transpose(selT, (0, 2, 1))

    ks = pl.pallas_call(
        _nms_body,
        grid=(B,),
        in_specs=[
            pl.BlockSpec((1, _K, 8), lambda i: (i, 0, 0)),
            pl.BlockSpec((1, 8, _K), lambda i: (i, 0, 0)),
        ],
        out_specs=pl.BlockSpec((1, 1, _K), lambda i: (i, 0, 0)),
        out_shape=jax.ShapeDtypeStruct((B, 1, _K), jnp.float32),
    )(sel, selT)[:, 0, :]

    det_scores, det_idx = jax.lax.top_k(ks, _MAXDET)
    det = jnp.take_along_axis(sel, det_idx[:, :, None], axis=1)
    mask = det_scores > (_NEG / 2)
    out = jnp.concatenate(
        [det[..., 0:4], det_scores[..., None], det[..., 5:6]], axis=2)
    out = jnp.where(mask[..., None], out, 0.0)
    counts = mask.sum(axis=1)
    return out, counts
